# baseline (speedup 1.0000x reference)
import jax
import jax.numpy as jnp
from jax.experimental import pallas as pl
from jax.experimental.pallas import tpu as pltpu

_B = 128
_NI = 1152
_NO = 10
_DO = 16
_DI = 8
_JK = _NO * _DO
_NUM_ROUTING = 3
_BC = 16


def _isum(t):
    ones = jnp.ones((_NI, 1), jnp.float32)
    r = jax.lax.dot_general(t.reshape(_BC * _JK, _NI), ones,
                            (((1,), (0,)), ((), ())),
                            precision=jax.lax.Precision.HIGHEST,
                            preferred_element_type=jnp.float32)
    return r.reshape(_BC, _NO, _DO)


def _kdot(u4, v):
    t = u4 * v[:, :, :, None]
    t8 = t[:, :, 0:8, :] + t[:, :, 8:16, :]
    return jnp.sum(t8, axis=2)


def _squash(s):
    sq = jnp.sum(s * s, axis=-1, keepdims=True)
    return (sq / (1.0 + sq)) * (s / jnp.sqrt(sq + 1e-9))


def _routing_body(xt_ref, wt_ref, out_ref, u_ref):
    x = xt_ref[...]
    acc = x[:, 0, None, :] * wt_ref[0][None, :, :]
    for d in range(1, _DI):
        acc = acc + x[:, d, None, :] * wt_ref[d][None, :, :]
    u_ref[...] = acc

    acc4 = acc.reshape(_BC, _NO, _DO, _NI)
    v = _squash(0.1 * _isum(acc4))

    u4 = u_ref[...].reshape(_BC, _NO, _DO, _NI)
    b_log = _kdot(u4, v)

    for r in range(1, _NUM_ROUTING):
        e = jnp.exp(b_log)
        c = e / (jnp.sum(e, axis=1, keepdims=True) + 1e-12)
        u4 = u_ref[...].reshape(_BC, _NO, _DO, _NI)
        v = _squash(_isum(u4 * c[:, :, None, :]))
        if r < _NUM_ROUTING - 1:
            b_log = b_log + _kdot(u4, v)
    out_ref[...] = v


def kernel(x, weight):
    xt = x.transpose(0, 2, 1)
    wt = weight.transpose(3, 1, 2, 0).reshape(_DI, _JK, _NI)
    return pl.pallas_call(
        _routing_body,
        grid=(_B // _BC,),
        in_specs=[
            pl.BlockSpec((_BC, _DI, _NI), lambda b: (b, 0, 0)),
            pl.BlockSpec((_DI, _JK, _NI), lambda b: (0, 0, 0)),
        ],
        out_specs=pl.BlockSpec((_BC, _NO, _DO), lambda b: (b, 0, 0)),
        out_shape=jax.ShapeDtypeStruct((_B, _NO, _DO), jnp.float32),
        scratch_shapes=[pltpu.VMEM((_BC, _JK, _NI), jnp.float32)],
    )(xt, wt)


# speedup vs baseline: 3.2688x; 3.2688x over previous
import jax
import jax.numpy as jnp
from jax.experimental import pallas as pl
from jax.experimental.pallas import tpu as pltpu

_B = 128
_NI = 1152
_NO = 10
_DO = 16
_DI = 8
_JK = _NO * _DO
_NUM_ROUTING = 3
_BC = 16


def _kdot(u_ref, v):
    t8 = (u_ref[:, :, 0:8, :] * v[:, :, 0:8, None]
          + u_ref[:, :, 8:16, :] * v[:, :, 8:16, None])
    return jnp.sum(t8, axis=2)


def _squash(s):
    sq = jnp.sum(s * s, axis=-1, keepdims=True)
    return (sq / (1.0 + sq)) * (s / jnp.sqrt(sq + 1e-9))


def _routing_body(xt_ref, wt_ref, out_ref, u_ref):
    x = xt_ref[...]
    acc = x[:, 0, None, :] * wt_ref[0][None, :, :]
    for d in range(1, _DI):
        acc = acc + x[:, d, None, :] * wt_ref[d][None, :, :]
    acc4 = acc.reshape(_BC, _NO, _DO, _NI)
    u_ref[...] = acc4

    v = _squash(0.1 * jnp.sum(acc4, axis=-1))

    b_log = _kdot(u_ref, v)

    for r in range(1, _NUM_ROUTING):
        e = jnp.exp(b_log)
        c = e / (jnp.sum(e, axis=1, keepdims=True) + 1e-12)
        v = _squash(jnp.sum(u_ref[...] * c[:, :, None, :], axis=-1))
        if r < _NUM_ROUTING - 1:
            b_log = b_log + _kdot(u_ref, v)
    out_ref[...] = v


def kernel(x, weight):
    xt = x.transpose(0, 2, 1)
    wt = weight.transpose(3, 1, 2, 0).reshape(_DI, _JK, _NI)
    return pl.pallas_call(
        _routing_body,
        grid=(_B // _BC,),
        in_specs=[
            pl.BlockSpec((_BC, _DI, _NI), lambda b: (b, 0, 0)),
            pl.BlockSpec((_DI, _JK, _NI), lambda b: (0, 0, 0)),
        ],
        out_specs=pl.BlockSpec((_BC, _NO, _DO), lambda b: (b, 0, 0)),
        out_shape=jax.ShapeDtypeStruct((_B, _NO, _DO), jnp.float32),
        scratch_shapes=[pltpu.VMEM((_BC, _NO, _DO, _NI), jnp.float32)],
    )(xt, wt)


# speedup vs baseline: 3.3280x; 1.0181x over previous
import jax
import jax.numpy as jnp
from jax.experimental import pallas as pl
from jax.experimental.pallas import tpu as pltpu

_B = 128
_NI = 1152
_NO = 10
_DO = 16
_DI = 8
_JK = _NO * _DO
_NUM_ROUTING = 3
_BC = 16


def _kdot(u4, v):
    return jnp.sum(u4 * v[:, :, :, None], axis=2)


def _squash(s):
    sq = jnp.sum(s * s, axis=-1, keepdims=True)
    return (sq / (1.0 + sq)) * (s / jnp.sqrt(sq + 1e-9))


def _routing_body(xt_ref, wt_ref, out_ref, u_ref):
    x = xt_ref[...]
    acc = x[:, 0, None, :] * wt_ref[0][None, :, :]
    for d in range(1, _DI):
        acc = acc + x[:, d, None, :] * wt_ref[d][None, :, :]
    acc4 = acc.reshape(_BC, _NO, _DO, _NI)
    u_ref[...] = acc4

    v = _squash(0.1 * jnp.sum(acc4, axis=-1))

    b_log = _kdot(u_ref[...], v)

    for r in range(1, _NUM_ROUTING):
        e = jnp.exp(b_log)
        c = e / (jnp.sum(e, axis=1, keepdims=True) + 1e-12)
        u4 = u_ref[...]
        v = _squash(jnp.sum(u4 * c[:, :, None, :], axis=-1))
        if r < _NUM_ROUTING - 1:
            b_log = b_log + _kdot(u4, v)
    out_ref[...] = v


def kernel(x, weight):
    xt = x.transpose(0, 2, 1)
    wt = weight.transpose(3, 1, 2, 0).reshape(_DI, _JK, _NI)
    return pl.pallas_call(
        _routing_body,
        grid=(_B // _BC,),
        in_specs=[
            pl.BlockSpec((_BC, _DI, _NI), lambda b: (b, 0, 0)),
            pl.BlockSpec((_DI, _JK, _NI), lambda b: (0, 0, 0)),
        ],
        out_specs=pl.BlockSpec((_BC, _NO, _DO), lambda b: (b, 0, 0)),
        out_shape=jax.ShapeDtypeStruct((_B, _NO, _DO), jnp.float32),
        scratch_shapes=[pltpu.VMEM((_BC, _NO, _DO, _NI), jnp.float32)],
    )(xt, wt)
